# baseline (speedup 1.0000x reference)
import functools

import jax
import jax.numpy as jnp
from jax import lax
from jax.experimental import pallas as pl
from jax.experimental.pallas import tpu as pltpu
from jax.experimental.pallas import tpu_sc as plsc

_E = 8
_R = 8
_TL = 1024
_LANES = 16
_NC, _NS = 2, 16


def _router_gates(cls, router_W):
    B, H = cls.shape
    E = router_W.shape[0]
    ER = E * _R
    mesh = plsc.VectorSubcoreMesh(
        core_axis_name="c", subcore_axis_name="s",
        num_cores=_NC, num_subcores=_NS)

    @functools.partial(
        pl.kernel,
        out_type=jax.ShapeDtypeStruct((B, ER), jnp.float32),
        mesh=mesh,
        compiler_params=pltpu.CompilerParams(needs_layout_passes=False),
        scratch_types=[
            pltpu.VMEM((H,), jnp.float32),
            pltpu.VMEM((E, H), jnp.float32),
            pltpu.VMEM((ER,), jnp.float32),
        ],
    )
    def _router(cls_hbm, rw_hbm, out_hbm, cls_v, rw_v, out_v):
        wid = lax.axis_index("s") * _NC + lax.axis_index("c")

        @pl.when(wid < B)
        def _():
            pltpu.sync_copy(cls_hbm.at[wid], cls_v)
            pltpu.sync_copy(rw_hbm, rw_v)

            iota = lax.iota(jnp.int32, _LANES)
            neg = jnp.float32(-1e30)
            logits = jnp.full((_LANES,), neg, jnp.float32)
            for e in range(E):
                def body(i, acc, e=e):
                    c = cls_v[pl.ds(i * _LANES, _LANES)]
                    w = rw_v[e, pl.ds(i * _LANES, _LANES)]
                    return acc + c * w
                acc = lax.fori_loop(0, H // _LANES, body,
                                    jnp.zeros((_LANES,), jnp.float32))
                logits = jnp.where(iota == e, jnp.sum(acc), logits)

            m1 = jnp.max(logits)
            i1 = jnp.min(jnp.where(logits == m1, iota, jnp.int32(2 * _LANES)))
            rest = jnp.where(iota == i1, neg, logits)
            m2 = jnp.max(rest)
            i2 = jnp.min(jnp.where(rest == m2, iota, jnp.int32(2 * _LANES)))
            in_top = (iota == i1) | (iota == i2)
            ex = jnp.where(in_top, jnp.exp(logits - m1), jnp.float32(0.0))
            gates = ex / jnp.sum(ex)

            for c4 in range(ER // _LANES):
                e_lo, e_hi = 2 * c4, 2 * c4 + 1
                g_lo = jnp.sum(jnp.where(iota == e_lo, gates, 0.0))
                g_hi = jnp.sum(jnp.where(iota == e_hi, gates, 0.0))
                out_v[pl.ds(c4 * _LANES, _LANES)] = jnp.where(
                    iota < _R, g_lo, g_hi)

            pltpu.sync_copy(out_v, out_hbm.at[wid])

    return _router(cls, router_W)


def _moe_body(x_ref, gr_ref, aall_ref, ball_ref, out_ref):
    b = pl.program_id(0)
    gates_rep = gr_ref[pl.ds(b, 1), :]
    xt = x_ref[0]
    down = jnp.dot(xt, aall_ref[...],
                   preferred_element_type=jnp.float32)
    down = down * gates_rep
    up = jnp.dot(down, ball_ref[...],
                 preferred_element_type=jnp.float32)
    out_ref[0] = xt + up


@jax.jit
def kernel(x, router_W, A, Bw):
    B, L, H = x.shape
    E, _, R = A.shape
    cls = x[:, 0, :]
    gates_rep = _router_gates(cls, router_W)

    A_all = A.transpose(1, 0, 2).reshape(H, E * R)
    B_all = Bw.reshape(E * R, H)

    grid = (B, L // _TL)
    return pl.pallas_call(
        _moe_body,
        grid=grid,
        in_specs=[
            pl.BlockSpec((1, _TL, H), lambda b, l: (b, l, 0)),
            pl.BlockSpec((B, E * R), lambda b, l: (0, 0)),
            pl.BlockSpec((H, E * R), lambda b, l: (0, 0)),
            pl.BlockSpec((E * R, H), lambda b, l: (0, 0)),
        ],
        out_specs=pl.BlockSpec((1, _TL, H), lambda b, l: (b, l, 0)),
        out_shape=jax.ShapeDtypeStruct((B, L, H), x.dtype),
    )(x, gates_rep, A_all, B_all)


# speedup vs baseline: 1.0413x; 1.0413x over previous
import functools

import jax
import jax.numpy as jnp
from jax import lax
from jax.experimental import pallas as pl
from jax.experimental.pallas import tpu as pltpu
from jax.experimental.pallas import tpu_sc as plsc

_E = 8
_R = 8
_TL = 1024
_LANES = 16
_NC, _NS = 2, 16


def _router_gates(x, router_W):
    B, _, H = x.shape
    E = router_W.shape[0]
    ER = E * _R
    mesh = plsc.VectorSubcoreMesh(
        core_axis_name="c", subcore_axis_name="s",
        num_cores=_NC, num_subcores=_NS)

    @functools.partial(
        pl.kernel,
        out_type=(
            jax.ShapeDtypeStruct((B, ER), jnp.float32),
            jax.ShapeDtypeStruct((_NC * _NS, _LANES), jnp.float32),
        ),
        mesh=mesh,
        compiler_params=pltpu.CompilerParams(needs_layout_passes=False),
        scratch_types=[
            pltpu.VMEM((H,), jnp.float32),
            pltpu.VMEM((H,), jnp.float32),
            pltpu.VMEM((E, _LANES), jnp.float32),
            pltpu.VMEM((ER,), jnp.float32),
        ],
    )
    def _router(x_hbm, rw_hbm, out_hbm, stage_hbm, cls_v, rw_v, loc_v, out_v):
        c = lax.axis_index("c")
        s = lax.axis_index("s")
        b = 2 * c + s // E
        e = s % E
        iota = lax.iota(jnp.int32, _LANES)
        neg = jnp.float32(-1e30)

        pltpu.sync_copy(x_hbm.at[b, 0], cls_v)
        pltpu.sync_copy(rw_hbm.at[e], rw_v)

        def body(i, accs):
            a0, a1, a2, a3 = accs
            o = i * (4 * _LANES)
            a0 += cls_v[pl.ds(o, _LANES)] * rw_v[pl.ds(o, _LANES)]
            a1 += (cls_v[pl.ds(o + _LANES, _LANES)]
                   * rw_v[pl.ds(o + _LANES, _LANES)])
            a2 += (cls_v[pl.ds(o + 2 * _LANES, _LANES)]
                   * rw_v[pl.ds(o + 2 * _LANES, _LANES)])
            a3 += (cls_v[pl.ds(o + 3 * _LANES, _LANES)]
                   * rw_v[pl.ds(o + 3 * _LANES, _LANES)])
            return a0, a1, a2, a3

        z = jnp.zeros((_LANES,), jnp.float32)
        a0, a1, a2, a3 = lax.fori_loop(0, H // (4 * _LANES), body,
                                       (z, z, z, z))
        logit = jnp.sum((a0 + a1) + (a2 + a3))
        loc_v[0, pl.ds(0, _LANES)] = jnp.zeros((_LANES,), jnp.float32) + logit
        pltpu.sync_copy(loc_v.at[0], stage_hbm.at[c * _NS + s])
        plsc.subcore_barrier()

        @pl.when(e == 0)
        def _():
            base = c * _NS + (s // E) * E
            pltpu.sync_copy(stage_hbm.at[pl.ds(base, E)], loc_v)
            logits = jnp.full((_LANES,), neg, jnp.float32)
            for j in range(E):
                row = loc_v[j, pl.ds(0, _LANES)]
                logits = jnp.where(iota == j, row, logits)

            m1 = jnp.max(logits)
            i1 = jnp.min(jnp.where(logits == m1, iota,
                                   jnp.int32(2 * _LANES)))
            rest = jnp.where(iota == i1, neg, logits)
            m2 = jnp.max(rest)
            i2 = jnp.min(jnp.where(rest == m2, iota,
                                   jnp.int32(2 * _LANES)))
            in_top = (iota == i1) | (iota == i2)
            ex = jnp.where(in_top, jnp.exp(logits - m1), jnp.float32(0.0))
            gates = ex / jnp.sum(ex)

            for c4 in range(ER // _LANES):
                e_lo, e_hi = 2 * c4, 2 * c4 + 1
                g_lo = jnp.sum(jnp.where(iota == e_lo, gates, 0.0))
                g_hi = jnp.sum(jnp.where(iota == e_hi, gates, 0.0))
                out_v[pl.ds(c4 * _LANES, _LANES)] = jnp.where(
                    iota < _R, g_lo, g_hi)

            pltpu.sync_copy(out_v, out_hbm.at[b])

    gates_rep, _ = _router(x, router_W)
    return gates_rep


def _moe_body(x_ref, gr_ref, aall_ref, ball_ref, out_ref):
    b = pl.program_id(0)
    gates_rep = gr_ref[pl.ds(b, 1), :]
    xt = x_ref[0]
    down = jnp.dot(xt, aall_ref[...],
                   preferred_element_type=jnp.float32)
    down = down * gates_rep
    up = jnp.dot(down, ball_ref[...],
                 preferred_element_type=jnp.float32)
    out_ref[0] = xt + up


@jax.jit
def kernel(x, router_W, A, Bw):
    B, L, H = x.shape
    E, _, R = A.shape
    gates_rep = _router_gates(x, router_W)

    A_all = A.transpose(1, 0, 2).reshape(H, E * R)
    B_all = Bw.reshape(E * R, H)

    grid = (B, L // _TL)
    return pl.pallas_call(
        _moe_body,
        grid=grid,
        in_specs=[
            pl.BlockSpec((1, _TL, H), lambda b, l: (b, l, 0)),
            pl.BlockSpec((B, E * R), lambda b, l: (0, 0)),
            pl.BlockSpec((H, E * R), lambda b, l: (0, 0)),
            pl.BlockSpec((E * R, H), lambda b, l: (0, 0)),
        ],
        out_specs=pl.BlockSpec((1, _TL, H), lambda b, l: (b, l, 0)),
        out_shape=jax.ShapeDtypeStruct((B, L, H), x.dtype),
    )(x, gates_rep, A_all, B_all)


# speedup vs baseline: 1.0469x; 1.0053x over previous
import functools

import jax
import jax.numpy as jnp
from jax import lax
from jax.experimental import pallas as pl
from jax.experimental.pallas import tpu as pltpu
from jax.experimental.pallas import tpu_sc as plsc

_E = 8
_R = 8
_TL = 1024
_LANES = 16
_NC, _NS = 2, 16


def _router_gates(x, router_W):
    B, _, H = x.shape
    E = router_W.shape[0]
    ER = E * _R
    mesh = plsc.VectorSubcoreMesh(
        core_axis_name="c", subcore_axis_name="s",
        num_cores=_NC, num_subcores=_NS)

    @functools.partial(
        pl.kernel,
        out_type=(
            jax.ShapeDtypeStruct((B, ER), jnp.float32),
            jax.ShapeDtypeStruct((_NC * _NS, _LANES), jnp.float32),
        ),
        mesh=mesh,
        compiler_params=pltpu.CompilerParams(needs_layout_passes=False),
        scratch_types=[
            pltpu.VMEM((H,), jnp.float32),
            pltpu.VMEM((H,), jnp.float32),
            pltpu.VMEM((E, _LANES), jnp.float32),
            pltpu.VMEM((ER,), jnp.float32),
        ],
    )
    def _router(x_hbm, rw_hbm, out_hbm, stage_hbm, cls_v, rw_v, loc_v, out_v):
        c = lax.axis_index("c")
        s = lax.axis_index("s")
        b = 2 * c + s // E
        e = s % E
        iota = lax.iota(jnp.int32, _LANES)
        neg = jnp.float32(-1e30)

        pltpu.sync_copy(x_hbm.at[b, 0], cls_v)
        pltpu.sync_copy(rw_hbm.at[e], rw_v)

        def body(i, accs):
            a0, a1, a2, a3 = accs
            o = i * (4 * _LANES)
            a0 += cls_v[pl.ds(o, _LANES)] * rw_v[pl.ds(o, _LANES)]
            a1 += (cls_v[pl.ds(o + _LANES, _LANES)]
                   * rw_v[pl.ds(o + _LANES, _LANES)])
            a2 += (cls_v[pl.ds(o + 2 * _LANES, _LANES)]
                   * rw_v[pl.ds(o + 2 * _LANES, _LANES)])
            a3 += (cls_v[pl.ds(o + 3 * _LANES, _LANES)]
                   * rw_v[pl.ds(o + 3 * _LANES, _LANES)])
            return a0, a1, a2, a3

        z = jnp.zeros((_LANES,), jnp.float32)
        a0, a1, a2, a3 = lax.fori_loop(0, H // (4 * _LANES), body,
                                       (z, z, z, z))
        logit = jnp.sum((a0 + a1) + (a2 + a3))
        loc_v[0, pl.ds(0, _LANES)] = jnp.zeros((_LANES,), jnp.float32) + logit
        pltpu.sync_copy(loc_v.at[0], stage_hbm.at[c * _NS + s])
        plsc.subcore_barrier()

        @pl.when(e == 0)
        def _():
            base = c * _NS + (s // E) * E
            pltpu.sync_copy(stage_hbm.at[pl.ds(base, E)], loc_v)
            logits = jnp.full((_LANES,), neg, jnp.float32)
            for j in range(E):
                row = loc_v[j, pl.ds(0, _LANES)]
                logits = jnp.where(iota == j, row, logits)

            m1 = jnp.max(logits)
            i1 = jnp.min(jnp.where(logits == m1, iota,
                                   jnp.int32(2 * _LANES)))
            rest = jnp.where(iota == i1, neg, logits)
            m2 = jnp.max(rest)
            i2 = jnp.min(jnp.where(rest == m2, iota,
                                   jnp.int32(2 * _LANES)))
            in_top = (iota == i1) | (iota == i2)
            ex = jnp.where(in_top, jnp.exp(logits - m1), jnp.float32(0.0))
            gates = ex / jnp.sum(ex)

            for c4 in range(ER // _LANES):
                e_lo, e_hi = 2 * c4, 2 * c4 + 1
                g_lo = jnp.sum(jnp.where(iota == e_lo, gates, 0.0))
                g_hi = jnp.sum(jnp.where(iota == e_hi, gates, 0.0))
                out_v[pl.ds(c4 * _LANES, _LANES)] = jnp.where(
                    iota < _R, g_lo, g_hi)

            pltpu.sync_copy(out_v, out_hbm.at[b])

    gates_rep, _ = _router(x, router_W)
    return gates_rep


def _moe_body(x_ref, gr_ref, aall_ref, ball_ref, out_ref):
    b = pl.program_id(0)
    gates_rep = gr_ref[pl.ds(b, 1), :]
    xt = x_ref[0]
    down = jnp.dot(xt.astype(jnp.bfloat16), aall_ref[...],
                   preferred_element_type=jnp.float32)
    down = (down * gates_rep).astype(jnp.bfloat16)
    up = jnp.dot(down, ball_ref[...],
                 preferred_element_type=jnp.float32)
    out_ref[0] = xt + up


@jax.jit
def kernel(x, router_W, A, Bw):
    B, L, H = x.shape
    E, _, R = A.shape
    gates_rep = _router_gates(x, router_W)

    A_all = A.transpose(1, 0, 2).reshape(H, E * R).astype(jnp.bfloat16)
    B_all = Bw.reshape(E * R, H).astype(jnp.bfloat16)

    grid = (B, L // _TL)
    return pl.pallas_call(
        _moe_body,
        grid=grid,
        in_specs=[
            pl.BlockSpec((1, _TL, H), lambda b, l: (b, l, 0)),
            pl.BlockSpec((B, E * R), lambda b, l: (0, 0)),
            pl.BlockSpec((H, E * R), lambda b, l: (0, 0)),
            pl.BlockSpec((E * R, H), lambda b, l: (0, 0)),
        ],
        out_specs=pl.BlockSpec((1, _TL, H), lambda b, l: (b, l, 0)),
        out_shape=jax.ShapeDtypeStruct((B, L, H), x.dtype),
    )(x, gates_rep, A_all, B_all)
